# baseline (speedup 1.0000x reference)
import functools

import jax
import jax.numpy as jnp
from jax import lax
from jax.experimental import pallas as pl
from jax.experimental.pallas import tpu as pltpu
from jax.experimental.pallas import tpu_sc as plsc

BATCH = 16384
VOCAB = 1000000
NF = 26
DIM = 10
PDIM = 16
D_IN = NF * DIM
PD_IN = NF * PDIM
TOTAL = BATCH * NF
SRC_W = 128
CHUNK = 128
NCHUNK = TOTAL // CHUNK
NWORKER = 32
CPT = NCHUNK // NWORKER
GRP = 8
NGRP = CPT // GRP


_sc_mesh = plsc.VectorSubcoreMesh(core_axis_name="c", subcore_axis_name="s")


@functools.partial(
    pl.kernel,
    out_type=(
        jax.ShapeDtypeStruct((NCHUNK, CHUNK, PDIM), jnp.float32),
        jax.ShapeDtypeStruct((NCHUNK, CHUNK), jnp.float32),
    ),
    mesh=_sc_mesh,
    compiler_params=pltpu.CompilerParams(use_tc_tiling_on_sc=False),
    scratch_types=[
        pltpu.VMEM((CPT, CHUNK), jnp.int32),
        pltpu.VMEM((CPT, CHUNK), jnp.int32),
        pltpu.VMEM((GRP, CHUNK, PDIM), jnp.float32),
        pltpu.VMEM((GRP, CHUNK), jnp.float32),
        pltpu.SemaphoreType.DMA,
        pltpu.SemaphoreType.DMA,
    ],
)
def _sc_gather(idx8_hbm, idx_hbm, deep_hbm, wsum_hbm, deep_out, wide_out,
               idx8_v, idx_v, dbuf, wbuf, dsem, wsem):
    wid = lax.axis_index("s") * 2 + lax.axis_index("c")
    c0 = wid * CPT
    pltpu.sync_copy(idx8_hbm.at[pl.ds(c0, CPT)], idx8_v)
    pltpu.sync_copy(idx_hbm.at[pl.ds(c0, CPT)], idx_v)

    for g in range(NGRP):
        base = g * GRP
        descs = []
        for j in range(GRP):
            descs.append(pltpu.async_copy(
                deep_hbm.at[idx8_v.at[base + j]], dbuf.at[j], dsem))
            descs.append(pltpu.async_copy(
                wsum_hbm.at[idx_v.at[base + j]], wbuf.at[j], wsem))
        for d in descs:
            d.wait()
        pltpu.sync_copy(dbuf, deep_out.at[pl.ds(c0 + base, GRP)])
        pltpu.sync_copy(wbuf, wide_out.at[pl.ds(c0 + base, GRP)])


PVB = 8192


def _prep_body(t_ref, e_ref, out_ref):
    out_ref[:, :PDIM] = lax.dot_general(
        t_ref[...], e_ref[...], (((0,), (0,)), ((), ())),
        precision=lax.Precision.HIGHEST,
        preferred_element_type=jnp.float32)


_prep = pl.pallas_call(
    _prep_body,
    grid=((VOCAB + PVB - 1) // PVB,),
    in_specs=[
        pl.BlockSpec((DIM, PVB), lambda i: (0, i)),
        pl.BlockSpec((DIM, PDIM), lambda i: (0, 0)),
    ],
    out_specs=pl.BlockSpec((PVB, SRC_W), lambda i: (i, 0)),
    out_shape=jax.ShapeDtypeStruct((VOCAB, SRC_W), jnp.float32),
    compiler_params=pltpu.CompilerParams(
        dimension_semantics=("arbitrary",),
    ),
)


VB = 65536


def _wsum_body(tbl_ref, out_ref):
    out_ref[...] = jnp.sum(tbl_ref[...], axis=0)


_wsum = pl.pallas_call(
    _wsum_body,
    grid=((VOCAB + VB - 1) // VB,),
    in_specs=[pl.BlockSpec((DIM, VB), lambda i: (0, i))],
    out_specs=pl.BlockSpec((VB,), lambda i: (i,)),
    out_shape=jax.ShapeDtypeStruct((VOCAB,), jnp.float32),
    compiler_params=pltpu.CompilerParams(
        dimension_semantics=("arbitrary",),
    ),
)


TB = 1024
INV_D = 1.0 / D_IN


def _dense_body(deep_ref, wide_ref, lns_ref, lnb_ref,
                W1_ref, b1_ref, W2_ref, b2_ref, W3_ref, b3_ref,
                Wm1_ref, bm1_ref, Wm2_ref, bm2_ref,
                Ws1_ref, bs1_ref, Ws2_ref, bs2_ref,
                mu_ref, ls_ref):
    d = deep_ref[...]
    mean = jnp.sum(d, axis=1, keepdims=True) * INV_D
    msq = jnp.sum(d * d, axis=1, keepdims=True) * INV_D
    var = msq - mean * mean
    h = (d - mean) * lax.rsqrt(var + 1e-5) * lns_ref[...] + lnb_ref[...]
    h = jnp.maximum(
        jnp.dot(h, W1_ref[...], preferred_element_type=jnp.float32) + b1_ref[...], 0.0)
    h = jnp.maximum(
        jnp.dot(h, W2_ref[...], preferred_element_type=jnp.float32) + b2_ref[...], 0.0)
    h = jnp.maximum(
        jnp.dot(h, W3_ref[...], preferred_element_type=jnp.float32) + b3_ref[...], 0.0)
    h = h + jnp.sum(wide_ref[...], axis=1, keepdims=True)
    m = jnp.maximum(
        jnp.dot(h, Wm1_ref[...], preferred_element_type=jnp.float32) + bm1_ref[...], 0.0)
    mu_ref[...] = jnp.dot(m, Wm2_ref[...], preferred_element_type=jnp.float32) + bm2_ref[...]
    s = jnp.maximum(
        jnp.dot(h, Ws1_ref[...], preferred_element_type=jnp.float32) + bs1_ref[...], 0.0)
    ls_ref[...] = jnp.dot(s, Ws2_ref[...], preferred_element_type=jnp.float32) + bs2_ref[...]


def _whole(shape):
    return pl.BlockSpec(shape, lambda i: tuple(0 for _ in shape))


_dense = pl.pallas_call(
    _dense_body,
    grid=(BATCH // TB,),
    in_specs=[
        pl.BlockSpec((TB, PD_IN), lambda i: (i, 0)),
        pl.BlockSpec((TB, NF), lambda i: (i, 0)),
        _whole((1, PD_IN)), _whole((1, PD_IN)),
        _whole((PD_IN, 512)), _whole((1, 512)),
        _whole((512, 256)), _whole((1, 256)),
        _whole((256, 64)), _whole((1, 64)),
        _whole((64, 32)), _whole((1, 32)),
        _whole((32, 1)), _whole((1, 1)),
        _whole((64, 32)), _whole((1, 32)),
        _whole((32, 1)), _whole((1, 1)),
    ],
    out_specs=[
        pl.BlockSpec((TB, 1), lambda i: (i, 0)),
        pl.BlockSpec((TB, 1), lambda i: (i, 0)),
    ],
    out_shape=[
        jax.ShapeDtypeStruct((BATCH, 1), jnp.float32),
        jax.ShapeDtypeStruct((BATCH, 1), jnp.float32),
    ],
    compiler_params=pltpu.CompilerParams(
        dimension_semantics=("arbitrary",),
    ),
)


def _pad_feat(a):
    a = a.reshape(NF, DIM, *a.shape[1:])
    pads = [(0, 0)] * a.ndim
    pads[1] = (0, PDIM - DIM)
    return jnp.pad(a, pads).reshape(NF * PDIM, *a.shape[2:])


def kernel(x, wide_table, deep_table, ln_scale, ln_bias,
           W1, b1, W2, b2, W3, b3,
           Wm1, bm1, Wm2, bm2, Ws1, bs1, Ws2, bs2):
    idx = x.reshape(NCHUNK, CHUNK)
    idx8 = (x * 8).reshape(NCHUNK, CHUNK)
    deep_lin = _prep(deep_table.T, jnp.eye(DIM, PDIM, dtype=jnp.float32))
    deep_rows = deep_lin.reshape(VOCAB * 8, PDIM)
    wsum = _wsum(wide_table.T)
    deep_g, wide_g = _sc_gather(idx8, idx, deep_rows, wsum)
    deep_emb = deep_g.reshape(BATCH, PD_IN)
    wide_part = wide_g.reshape(BATCH, NF)
    mu, ls = _dense(
        deep_emb, wide_part,
        _pad_feat(ln_scale).reshape(1, PD_IN), _pad_feat(ln_bias).reshape(1, PD_IN),
        _pad_feat(W1), b1.reshape(1, 512), W2, b2.reshape(1, 256), W3, b3.reshape(1, 64),
        Wm1, bm1.reshape(1, 32), Wm2, bm2.reshape(1, 1),
        Ws1, bs1.reshape(1, 32), Ws2, bs2.reshape(1, 1),
    )
    return mu.reshape(BATCH), ls.reshape(BATCH)


# speedup vs baseline: 1.4930x; 1.4930x over previous
import functools

import jax
import jax.numpy as jnp
from jax import lax
from jax.experimental import pallas as pl
from jax.experimental.pallas import tpu as pltpu
from jax.experimental.pallas import tpu_sc as plsc

BATCH = 16384
VOCAB = 1000000
NF = 26
DIM = 10
PDIM = 16
D_IN = NF * DIM
PD_IN = NF * PDIM
TOTAL = BATCH * NF
SRC_W = 128
CHUNK = 128
NCHUNK = TOTAL // CHUNK
NWORKER = 32
CPT = NCHUNK // NWORKER
GRP = 8
NGRP = CPT // GRP


_sc_mesh = plsc.VectorSubcoreMesh(core_axis_name="c", subcore_axis_name="s")


@functools.partial(
    pl.kernel,
    out_type=(
        jax.ShapeDtypeStruct((NCHUNK, CHUNK, PDIM), jnp.float32),
        jax.ShapeDtypeStruct((NCHUNK, CHUNK), jnp.float32),
    ),
    mesh=_sc_mesh,
    compiler_params=pltpu.CompilerParams(use_tc_tiling_on_sc=False),
    scratch_types=[
        pltpu.VMEM((CPT, CHUNK), jnp.int32),
        pltpu.VMEM((CPT, CHUNK), jnp.int32),
        pltpu.VMEM((GRP, CHUNK, PDIM), jnp.float32),
        pltpu.VMEM((GRP, CHUNK), jnp.float32),
        pltpu.SemaphoreType.DMA,
        pltpu.SemaphoreType.DMA,
    ],
)
def _sc_gather(idx8_hbm, idx_hbm, deep_hbm, wsum_hbm, deep_out, wide_out,
               idx8_v, idx_v, dbuf, wbuf, dsem, wsem):
    wid = lax.axis_index("s") * 2 + lax.axis_index("c")
    c0 = wid * CPT
    pltpu.sync_copy(idx8_hbm.at[pl.ds(c0, CPT)], idx8_v)
    pltpu.sync_copy(idx_hbm.at[pl.ds(c0, CPT)], idx_v)

    for g in range(NGRP):
        base = g * GRP
        descs = []
        for j in range(GRP):
            descs.append(pltpu.async_copy(
                deep_hbm.at[idx8_v.at[base + j]], dbuf.at[j], dsem))
            descs.append(pltpu.async_copy(
                wsum_hbm.at[idx_v.at[base + j]], wbuf.at[j], wsem))
        for d in descs:
            d.wait()
        pltpu.sync_copy(dbuf, deep_out.at[pl.ds(c0 + base, GRP)])
        pltpu.sync_copy(wbuf, wide_out.at[pl.ds(c0 + base, GRP)])


PVB = 8192


def _prep_body(t_ref, e_ref, out_ref):
    del e_ref
    out_ref[:, :DIM] = jnp.transpose(t_ref[...])
    out_ref[:, DIM:PDIM] = jnp.zeros((PVB, PDIM - DIM), jnp.float32)


_prep = pl.pallas_call(
    _prep_body,
    grid=((VOCAB + PVB - 1) // PVB,),
    in_specs=[
        pl.BlockSpec((DIM, PVB), lambda i: (0, i)),
        pl.BlockSpec((DIM, PDIM), lambda i: (0, 0)),
    ],
    out_specs=pl.BlockSpec((PVB, SRC_W), lambda i: (i, 0)),
    out_shape=jax.ShapeDtypeStruct((VOCAB, SRC_W), jnp.float32),
    compiler_params=pltpu.CompilerParams(
        dimension_semantics=("arbitrary",),
    ),
)


VB = 65536


def _wsum_body(tbl_ref, out_ref):
    out_ref[...] = jnp.sum(tbl_ref[...], axis=0)


_wsum = pl.pallas_call(
    _wsum_body,
    grid=((VOCAB + VB - 1) // VB,),
    in_specs=[pl.BlockSpec((DIM, VB), lambda i: (0, i))],
    out_specs=pl.BlockSpec((VB,), lambda i: (i,)),
    out_shape=jax.ShapeDtypeStruct((VOCAB,), jnp.float32),
    compiler_params=pltpu.CompilerParams(
        dimension_semantics=("arbitrary",),
    ),
)


TB = 1024
INV_D = 1.0 / D_IN


def _dense_body(deep_ref, wide_ref, lns_ref, lnb_ref,
                W1_ref, b1_ref, W2_ref, b2_ref, W3_ref, b3_ref,
                Wm1_ref, bm1_ref, Wm2_ref, bm2_ref,
                Ws1_ref, bs1_ref, Ws2_ref, bs2_ref,
                mu_ref, ls_ref):
    d = deep_ref[...]
    mean = jnp.sum(d, axis=1, keepdims=True) * INV_D
    msq = jnp.sum(d * d, axis=1, keepdims=True) * INV_D
    var = msq - mean * mean
    h = (d - mean) * lax.rsqrt(var + 1e-5) * lns_ref[...] + lnb_ref[...]
    h = jnp.maximum(
        jnp.dot(h, W1_ref[...], preferred_element_type=jnp.float32) + b1_ref[...], 0.0)
    h = jnp.maximum(
        jnp.dot(h, W2_ref[...], preferred_element_type=jnp.float32) + b2_ref[...], 0.0)
    h = jnp.maximum(
        jnp.dot(h, W3_ref[...], preferred_element_type=jnp.float32) + b3_ref[...], 0.0)
    h = h + jnp.sum(wide_ref[...], axis=1, keepdims=True)
    m = jnp.maximum(
        jnp.dot(h, Wm1_ref[...], preferred_element_type=jnp.float32) + bm1_ref[...], 0.0)
    mu_ref[...] = jnp.dot(m, Wm2_ref[...], preferred_element_type=jnp.float32) + bm2_ref[...]
    s = jnp.maximum(
        jnp.dot(h, Ws1_ref[...], preferred_element_type=jnp.float32) + bs1_ref[...], 0.0)
    ls_ref[...] = jnp.dot(s, Ws2_ref[...], preferred_element_type=jnp.float32) + bs2_ref[...]


def _whole(shape):
    return pl.BlockSpec(shape, lambda i: tuple(0 for _ in shape))


_dense = pl.pallas_call(
    _dense_body,
    grid=(BATCH // TB,),
    in_specs=[
        pl.BlockSpec((TB, PD_IN), lambda i: (i, 0)),
        pl.BlockSpec((TB, NF), lambda i: (i, 0)),
        _whole((1, PD_IN)), _whole((1, PD_IN)),
        _whole((PD_IN, 512)), _whole((1, 512)),
        _whole((512, 256)), _whole((1, 256)),
        _whole((256, 64)), _whole((1, 64)),
        _whole((64, 32)), _whole((1, 32)),
        _whole((32, 1)), _whole((1, 1)),
        _whole((64, 32)), _whole((1, 32)),
        _whole((32, 1)), _whole((1, 1)),
    ],
    out_specs=[
        pl.BlockSpec((TB, 1), lambda i: (i, 0)),
        pl.BlockSpec((TB, 1), lambda i: (i, 0)),
    ],
    out_shape=[
        jax.ShapeDtypeStruct((BATCH, 1), jnp.float32),
        jax.ShapeDtypeStruct((BATCH, 1), jnp.float32),
    ],
    compiler_params=pltpu.CompilerParams(
        dimension_semantics=("arbitrary",),
    ),
)


def _pad_feat(a):
    a = a.reshape(NF, DIM, *a.shape[1:])
    pads = [(0, 0)] * a.ndim
    pads[1] = (0, PDIM - DIM)
    return jnp.pad(a, pads).reshape(NF * PDIM, *a.shape[2:])


def kernel(x, wide_table, deep_table, ln_scale, ln_bias,
           W1, b1, W2, b2, W3, b3,
           Wm1, bm1, Wm2, bm2, Ws1, bs1, Ws2, bs2):
    idx = x.reshape(NCHUNK, CHUNK)
    idx8 = (x * 8).reshape(NCHUNK, CHUNK)
    deep_lin = _prep(deep_table.T, jnp.eye(DIM, PDIM, dtype=jnp.float32))
    deep_rows = deep_lin.reshape(VOCAB * 8, PDIM)
    wsum = _wsum(wide_table.T)
    deep_g, wide_g = _sc_gather(idx8, idx, deep_rows, wsum)
    deep_emb = deep_g.reshape(BATCH, PD_IN)
    wide_part = wide_g.reshape(BATCH, NF)
    mu, ls = _dense(
        deep_emb, wide_part,
        _pad_feat(ln_scale).reshape(1, PD_IN), _pad_feat(ln_bias).reshape(1, PD_IN),
        _pad_feat(W1), b1.reshape(1, 512), W2, b2.reshape(1, 256), W3, b3.reshape(1, 64),
        Wm1, bm1.reshape(1, 32), Wm2, bm2.reshape(1, 1),
        Ws1, bs1.reshape(1, 32), Ws2, bs2.reshape(1, 1),
    )
    return mu.reshape(BATCH), ls.reshape(BATCH)
